# Initial kernel scaffold; baseline (speedup 1.0000x reference)
import jax
import jax.numpy as jnp
from jax.experimental import pallas as pl


def kernel(x, edge_index, edge_attr, batch, params):
    raise NotImplementedError("write your pallas kernel here")


# speedup vs baseline: 9.1645x; 9.1645x over previous
import dataclasses
import functools

import jax
import jax.numpy as jnp
from jax import lax
from jax.experimental import pallas as pl
from jax.experimental.pallas import tpu as pltpu
from jax.experimental.pallas import tpu_sc as plsc

N = 10000
E = 320000
IN_DIM = 128
HID = 128
HEADS = 8
CH = HID // HEADS
G = 128

NC = 2
NS = 16
NW = NC * NS
EPT = E // NW
BE = 40
NBLKE = EPT // BE
QSTW = 160
KVW = 256
DAW = 16
NROW = 10112
RPT = NROW // NS
NROW8 = 1280
RPT8 = NROW8 // NS
BEP = BE + 8

NBLK = 10


def _gelu_p(x):
    return 0.5 * x * (1.0 + jax.lax.erf(x * 0.7071067811865476))


def _ln_p(h, g, b):
    m = jnp.mean(h, axis=-1, keepdims=True)
    v = jnp.mean((h - m) * (h - m), axis=-1, keepdims=True)
    return (h - m) * jax.lax.rsqrt(v + 1e-5) * g + b


def _in_proj_kernel(x_ref, w_ref, b_ref, g_ref, be_ref, o_ref):
    h = jnp.dot(x_ref[...], w_ref[...], preferred_element_type=jnp.float32)
    o_ref[...] = _ln_p(_gelu_p(h + b_ref[...]), g_ref[...], be_ref[...])


def _in_proj(x, W, b, g, be):
    blk = N // NBLK
    return pl.pallas_call(
        _in_proj_kernel,
        grid=(NBLK,),
        in_specs=[
            pl.BlockSpec((blk, IN_DIM), lambda i: (i, 0)),
            pl.BlockSpec((IN_DIM, HID), lambda i: (0, 0)),
            pl.BlockSpec((1, HID), lambda i: (0, 0)),
            pl.BlockSpec((1, HID), lambda i: (0, 0)),
            pl.BlockSpec((1, HID), lambda i: (0, 0)),
        ],
        out_specs=pl.BlockSpec((blk, HID), lambda i: (i, 0)),
        out_shape=jax.ShapeDtypeStruct((N, HID), jnp.float32),
    )(x, W, b.reshape(1, HID), g.reshape(1, HID), be.reshape(1, HID))


def _pre_kernel(h_ref, wq_ref, bq_ref, wkv_ref, bkv_ref, ws_ref, bs_ref,
                q_ref, kv_ref, xr_ref):
    h = h_ref[...]
    q_ref[...] = jnp.dot(h, wq_ref[...], preferred_element_type=jnp.float32) + bq_ref[...]
    kv_ref[...] = jnp.dot(h, wkv_ref[...], preferred_element_type=jnp.float32) + bkv_ref[...]
    xr_ref[...] = jnp.dot(h, ws_ref[...], preferred_element_type=jnp.float32) + bs_ref[...]


def _pre(h, Wq, bq, Wkv, bkv, Ws, bs):
    blk = N // NBLK
    return pl.pallas_call(
        _pre_kernel,
        grid=(NBLK,),
        in_specs=[
            pl.BlockSpec((blk, HID), lambda i: (i, 0)),
            pl.BlockSpec((HID, HID), lambda i: (0, 0)),
            pl.BlockSpec((1, HID), lambda i: (0, 0)),
            pl.BlockSpec((HID, KVW), lambda i: (0, 0)),
            pl.BlockSpec((1, KVW), lambda i: (0, 0)),
            pl.BlockSpec((HID, HID), lambda i: (0, 0)),
            pl.BlockSpec((1, HID), lambda i: (0, 0)),
        ],
        out_specs=[
            pl.BlockSpec((blk, HID), lambda i: (i, 0)),
            pl.BlockSpec((blk, KVW), lambda i: (i, 0)),
            pl.BlockSpec((blk, HID), lambda i: (i, 0)),
        ],
        out_shape=[
            jax.ShapeDtypeStruct((N, HID), jnp.float32),
            jax.ShapeDtypeStruct((N, KVW), jnp.float32),
            jax.ShapeDtypeStruct((N, HID), jnp.float32),
        ],
    )(h, Wq, bq.reshape(1, HID), Wkv, bkv.reshape(1, KVW), Ws,
      bs.reshape(1, HID))


def _post_kernel(m0_ref, m1_ref, d0_ref, d1_ref, xr_ref, hres_ref,
                 rep_ref, wbo_ref, wbx_ref, g_ref, bn_ref,
                 o_ref):
    m = m0_ref[0] + m1_ref[0]
    dA = d0_ref[0] + d1_ref[0]
    den8 = dA[:, 0:8]
    denx = jnp.dot(den8, rep_ref[...], preferred_element_type=jnp.float32) + 1e-16
    out = m / denx
    xr = xr_ref[...]
    z = (jnp.dot(out, wbo_ref[...], preferred_element_type=jnp.float32)
         + jnp.dot(xr, wbx_ref[...], preferred_element_type=jnp.float32))
    bg = jax.nn.sigmoid(z)
    hc = bg * xr + (1.0 - bg) * out
    o_ref[...] = _ln_p(_gelu_p(hc) + hres_ref[...], g_ref[...], bn_ref[...])


def _post(msg, denA, xr, hres, Rep, Wbo, Wbx, g, bn):
    blk = N // NBLK
    return pl.pallas_call(
        _post_kernel,
        grid=(NBLK,),
        in_specs=[
            pl.BlockSpec((1, blk, HID), lambda i: (0, i, 0)),
            pl.BlockSpec((1, blk, HID), lambda i: (1, i, 0)),
            pl.BlockSpec((1, blk, DAW), lambda i: (0, i, 0)),
            pl.BlockSpec((1, blk, DAW), lambda i: (1, i, 0)),
            pl.BlockSpec((blk, HID), lambda i: (i, 0)),
            pl.BlockSpec((blk, HID), lambda i: (i, 0)),
            pl.BlockSpec((HEADS, HID), lambda i: (0, 0)),
            pl.BlockSpec((HID, 128), lambda i: (0, 0)),
            pl.BlockSpec((HID, 128), lambda i: (0, 0)),
            pl.BlockSpec((1, HID), lambda i: (0, 0)),
            pl.BlockSpec((1, HID), lambda i: (0, 0)),
        ],
        out_specs=pl.BlockSpec((blk, HID), lambda i: (i, 0)),
        out_shape=jax.ShapeDtypeStruct((N, HID), jnp.float32),
    )(msg, msg, denA, denA, xr, hres, Rep, Wbo, Wbx,
      g.reshape(1, HID), bn.reshape(1, HID))


_MESH = plsc.VectorSubcoreMesh(core_axis_name="c", subcore_axis_name="s")

_SC_PARAMS = pltpu.CompilerParams()
if "needs_layout_passes" in pltpu.CompilerParams.__dataclass_fields__:
    _SC_PARAMS = dataclasses.replace(_SC_PARAMS, needs_layout_passes=False)


ERP = 2560
RPTE = ERP // NW
HALF = 5056
HR = 5120
HR8 = 640


def _edge_pass(q, kv, srcp, dstp, easp, doffp, ebc):
    @functools.partial(
        pl.kernel,
        out_type=[
            jax.ShapeDtypeStruct((NC, 2, HR, HID), jnp.float32),
            jax.ShapeDtypeStruct((NC, 2, HR8, HID), jnp.float32),
        ],
        mesh=_MESH,
        compiler_params=_SC_PARAMS,
        scratch_types=[
            pltpu.VMEM((64, HID), jnp.float32),
            pltpu.VMEM((64, KVW), jnp.float32),
            pltpu.VMEM((128, HID), jnp.float32),
            pltpu.VMEM((128, HID), jnp.float32),
            pltpu.VMEM((8, 128), jnp.int32),
            pltpu.VMEM((8, 128), jnp.int32),
            pltpu.VMEM((8, 128), jnp.int32),
            pltpu.VMEM((8, 128), jnp.int32),
            pltpu.VMEM((16, 128), jnp.float32),
            pltpu.VMEM((16, 128), jnp.int32),
            pltpu.VMEM((8, HID), jnp.float32),
            pltpu.VMEM_SHARED((HR, HID), jnp.float32),
            pltpu.VMEM_SHARED((HR8, HID), jnp.float32),
        ],
    )
    def k(q_hbm, kv_hbm, src_hbm, dst_hbm, easp_hbm, doffp_hbm, ebc_hbm,
          msg_hbm, da_hbm, qv, kvv, msgv, dav, srcv, dstv, dstR, d8R,
          eav, dov, ebcv, msh, dsh):
        cid = lax.axis_index("c")
        sid = lax.axis_index("s")
        wid = cid * NS + sid
        iota = lax.iota(jnp.int32, 16)
        pltpu.sync_copy(ebc_hbm, ebcv)

        for p in range(2):
            lo = p * HALF
            lo8 = p * (HALF // 8)
            @pl.loop(0, 128)
            def _zrow(r):
                for c in range(HID // 16):
                    msgv[r, pl.ds(16 * c, 16)] = jnp.zeros((16,), jnp.float32)

            row0 = sid * (HR // NS)
            pltpu.sync_copy(msgv, msh.at[pl.ds(row0, 128)])
            pltpu.sync_copy(msgv, msh.at[pl.ds(row0 + 128, 128)])
            pltpu.sync_copy(msgv.at[pl.ds(0, 64)], msh.at[pl.ds(row0 + 256, 64)])
            pltpu.sync_copy(msgv.at[pl.ds(0, 40)], dsh.at[pl.ds(sid * (HR8 // NS), 40)])
            plsc.subcore_barrier()

            @pl.loop(0, RPTE // 8)
            def _chunk(cix):
                rowbase = wid * RPTE + cix * 8
                pltpu.sync_copy(src_hbm.at[pl.ds(rowbase, 8)], srcv)
                pltpu.sync_copy(dst_hbm.at[pl.ds(rowbase, 8)], dstv)
                for r in range(8):
                    for t in range(8):
                        d = dstv[r, pl.ds(16 * t, 16)]
                        inr = (d >= lo) & (d < lo + HALF)
                        dstR[r, pl.ds(16 * t, 16)] = jnp.where(inr, d - lo, HR - 1)
                        d8 = lax.shift_right_logical(d, 3)
                        d8R[r, pl.ds(16 * t, 16)] = jnp.where(inr, d8 - lo8, HR8 - 1)
                for r in range(8):
                    erow = (rowbase + r) * 16
                    pltpu.sync_copy(easp_hbm.at[pl.ds(erow, 16)], eav)
                    pltpu.sync_copy(doffp_hbm.at[pl.ds(erow, 16)], dov)
                    for s2 in range(2):
                        pltpu.sync_copy(kv_hbm.at[srcv.at[r, pl.ds(64 * s2, 64)]], kvv)
                        pltpu.sync_copy(q_hbm.at[dstv.at[r, pl.ds(64 * s2, 64)]], qv)

                        @pl.loop(0, 64)
                        def _edge(i):
                            r8 = lax.shift_right_logical(i, 3) + 8 * s2
                            off = pl.multiple_of((i & 7) * 16, 16)
                            ear = eav[r8, pl.ds(off, 16)]
                            acc = jnp.zeros((16,), jnp.float32)
                            for h in range(HEADS):
                                er = ear * ebcv[0, pl.ds(16 * h, 16)] + ebcv[1, pl.ds(16 * h, 16)]
                                qr = qv[i, pl.ds(16 * h, 16)]
                                kr = kvv[i, pl.ds(16 * h, 16)] + er
                                s = jnp.sum(qr * kr)
                                acc = acc + jnp.where(iota == h, s, 0.0)
                            exv = jnp.exp(acc * 0.25)
                            dvec = dov[r8, pl.ds(off, 16)]
                            for c in range(HID // 16):
                                dav[i + 64 * s2, pl.ds(16 * c, 16)] = jnp.where(
                                    dvec == 16 * c, exv, 0.0)
                            for h in range(HEADS):
                                er = ear * ebcv[0, pl.ds(16 * h, 16)] + ebcv[1, pl.ds(16 * h, 16)]
                                sh = jnp.sum(jnp.where(iota == h, exv, 0.0))
                                msgv[i + 64 * s2, pl.ds(16 * h, 16)] = (
                                    kvv[i, pl.ds(128 + 16 * h, 16)] + er) * sh

                    pltpu.sync_copy(msgv, msh.at[dstR.at[r]], add=True)
                    pltpu.sync_copy(dav, dsh.at[d8R.at[r]], add=True)

            plsc.subcore_barrier()

            pltpu.sync_copy(msh.at[pl.ds(row0, 128)], msgv)
            pltpu.sync_copy(msgv, msg_hbm.at[cid, p, pl.ds(row0, 128)])
            pltpu.sync_copy(msh.at[pl.ds(row0 + 128, 128)], msgv)
            pltpu.sync_copy(msgv, msg_hbm.at[cid, p, pl.ds(row0 + 128, 128)])
            pltpu.sync_copy(msh.at[pl.ds(row0 + 256, 64)], msgv.at[pl.ds(0, 64)])
            pltpu.sync_copy(msgv.at[pl.ds(0, 64)], msg_hbm.at[cid, p, pl.ds(row0 + 256, 64)])
            pltpu.sync_copy(dsh.at[pl.ds(sid * (HR8 // NS), 40)], dav.at[pl.ds(0, 40)])
            pltpu.sync_copy(dav.at[pl.ds(0, 40)],
                            da_hbm.at[cid, p, pl.ds(sid * (HR8 // NS), 40)])
            if p == 0:
                plsc.subcore_barrier()

    return k(q, kv, srcp, dstp, easp, doffp, ebc)


def _head_mask():
    import numpy as np
    j = np.arange(HID)
    m = (j[None, :] // CH == np.arange(HEADS)[:, None]).astype(np.float32)
    return jnp.asarray(m)


def _layer_weights(lp):
    M = _head_mask()
    Wkv = jnp.concatenate([lp['Wk'], lp['Wv']], axis=1)
    bkv = jnp.concatenate([lp['bk'], lp['bv']])
    ebc = jnp.concatenate([jnp.stack([lp['We'].reshape(HID), lp['be']]),
                           jnp.zeros((6, HID), jnp.float32)])
    Rep = M
    Wb = lp['Wb']
    Wbo = jnp.tile(Wb[0:HID] + Wb[2 * HID:3 * HID], (1, 128))
    Wbx = jnp.tile(Wb[HID:2 * HID] - Wb[2 * HID:3 * HID], (1, 128))
    return Wkv, bkv, ebc, Rep, Wbo, Wbx


def kernel(x, edge_index, edge_attr, batch, params):
    epad = ERP * 128 - E
    srcp = jnp.concatenate([edge_index[0], jnp.zeros((epad,), jnp.int32)]).reshape(ERP, 128)
    dstf = jnp.concatenate([edge_index[1], jnp.full((epad,), 10111, jnp.int32)])
    dstp = dstf.reshape(ERP, 128)
    eaf = jnp.concatenate([edge_attr.reshape(E), jnp.zeros((epad,), jnp.float32)])
    easp = jnp.repeat(eaf.reshape(ERP * 16, 8), 16, axis=1)
    doffp = jnp.repeat(((dstf & 7) * 16).reshape(ERP * 16, 8), 16, axis=1)
    h = _in_proj(x, params['W_in'], params['b_in'], params['g_in'],
                 params['be_in'])
    for lp in params['layers']:
        Wkv, bkv, ebc, Rep, Wbo, Wbx = _layer_weights(lp)
        q, kv, xr = _pre(h, lp['Wq'], lp['bq'], Wkv, bkv, lp['Ws'], lp['bs'])
        msgh, dah = _edge_pass(q, kv, srcp, dstp, easp, doffp, ebc)
        msg = jnp.concatenate([msgh[:, 0, :HALF], msgh[:, 1, :HALF]], axis=1)
        den2 = jnp.concatenate([dah[:, 0, :HALF // 8], dah[:, 1, :HALF // 8]], axis=1)
        denA = den2.reshape(NC, (HALF // 4) * HID)[:, :N * DAW].reshape(NC, N, DAW)
        h = _post(msg, denA, xr, h, Rep, Wbo, Wbx, lp['g'], lp['bn'])
    s = jax.ops.segment_sum(h, batch, num_segments=G)
    cnt = jax.ops.segment_sum(jnp.ones((h.shape[0], 1), h.dtype), batch,
                              num_segments=G)
    gmean = s / jnp.maximum(cnt, 1.0)
    gmax = jax.ops.segment_max(h, batch, num_segments=G)
    gmax = jnp.where(jnp.isfinite(gmax), gmax, 0.0)
    gr = jnp.concatenate([gmean, gmax], axis=-1)
    o = jax.nn.gelu(gr @ params['W1'] + params['b1'], approximate=False)
    o = jax.nn.gelu(o @ params['W2'] + params['b2'], approximate=False)
    o = o @ params['W3'] + params['b3']
    return o.reshape(-1)
